# baseline (speedup 1.0000x reference)
import functools

import jax
import jax.numpy as jnp
from jax import lax
from jax.experimental import pallas as pl
from jax.experimental.pallas import tpu as pltpu
from jax.experimental.pallas import tpu_sc as plsc

D = 64
NU = 4096
NI = 4096
T = 16384
U = 1024
IF_LR = 0.01
L = 16
NC = 2
NS = 16
NW = NC * NS
C = 128
CHUNKS = 5
P_TOT = NW * CHUNKS * C
ROWS_PER_TILE = NU // NS

_C1T = -IF_LR / (T * T)
_C1U = IF_LR / T


def _sc_body(ue, ie, pp, ia, ib, yh, kh, acc_out,
             idxA_v, idxB_v, paidx_v, pbidx_v, saidx_v, sbidx_v,
             y_v, k_v, s_v, ds_v, c1_v, c2_v, A_v, B_v, PA_v, PB_v, CU_v, CI_v,
             accu_sh, acci_sh, sem):
    cid = lax.axis_index("c")
    sid = lax.axis_index("s")
    wid = sid * NC + cid

    def _zero_row(i, _):
        q = i % 4
        r = i // 4
        CU_v[r, pl.ds(q * L, L)] = jnp.zeros((L,), jnp.float32)
        return 0
    lax.fori_loop(0, C * 4, _zero_row, 0)
    pltpu.sync_copy(CU_v, accu_sh.at[pl.ds(sid * ROWS_PER_TILE, C)])
    pltpu.sync_copy(CU_v, accu_sh.at[pl.ds(sid * ROWS_PER_TILE + C, C)])
    pltpu.sync_copy(CU_v, acci_sh.at[pl.ds(sid * ROWS_PER_TILE, C)])
    pltpu.sync_copy(CU_v, acci_sh.at[pl.ds(sid * ROWS_PER_TILE + C, C)])
    plsc.subcore_barrier()

    def _chunk(g, _):
        base = wid * (CHUNKS * C) + g * C
        pltpu.sync_copy(ia.at[pl.ds(base, C)], idxA_v)
        pltpu.sync_copy(ib.at[pl.ds(base, C)], idxB_v)
        pltpu.sync_copy(yh.at[pl.ds(base, C)], y_v)
        pltpu.sync_copy(kh.at[pl.ds(base, C)], k_v)

        def _idx(q, _):
            va = idxA_v[pl.ds(q * L, L)]
            vb = idxB_v[pl.ds(q * L, L)]
            paidx_v[pl.ds(q * L, L)] = jnp.where(va < NU, va, 2 * NU)
            pbidx_v[pl.ds(q * L, L)] = jnp.where(vb < NI, vb + NU, 2 * NU)
            saidx_v[pl.ds(q * L, L)] = jnp.minimum(va, NU)
            sbidx_v[pl.ds(q * L, L)] = jnp.minimum(vb, NI)
            return 0
        lax.fori_loop(0, C // L, _idx, 0)

        cpa = pltpu.make_async_copy(ue.at[idxA_v], A_v, sem)
        cpb = pltpu.make_async_copy(ie.at[idxB_v], B_v, sem)
        cpp = pltpu.make_async_copy(pp.at[paidx_v], PA_v, sem)
        cpq = pltpu.make_async_copy(pp.at[pbidx_v], PB_v, sem)
        cpa.start(); cpb.start(); cpp.start(); cpq.start()
        cpa.wait(); cpb.wait(); cpp.wait(); cpq.wait()

        def _zero16(q, _):
            s_v[pl.ds(q * L, L)] = jnp.zeros((L,), jnp.float32)
            ds_v[pl.ds(q * L, L)] = jnp.zeros((L,), jnp.float32)
            return 0
        lax.fori_loop(0, C // L, _zero16, 0)

        def _dots(i, _):
            sv = jnp.zeros((L,), jnp.float32)
            dv = jnp.zeros((L,), jnp.float32)
            for q in range(D // L):
                sl = pl.ds(q * L, L)
                a = A_v[i, sl]
                b = B_v[i, sl]
                sv = sv + a * b
                dv = dv + b * PA_v[i, sl] + a * PB_v[i, sl]
            lane_i = jnp.full((L,), i, jnp.int32)
            plsc.addupdate_scatter(s_v, [lane_i], sv)
            plsc.addupdate_scatter(ds_v, [lane_i], dv)
            return 0
        lax.fori_loop(0, C, _dots, 0)

        def _group(q, _):
            sl = pl.ds(q * L, L)
            s = s_v[sl]
            ds = ds_v[sl]
            y = y_v[sl]
            k = k_v[sl]
            sig = 1.0 / (1.0 + jnp.exp(-s))
            lp = sig - y
            lpp = sig * (1.0 - sig)
            c1t = _C1T * ds * lpp
            c1_v[sl] = k * (_C1U * lp) + (1.0 - k) * c1t
            c2_v[sl] = (1.0 - k) * (_C1T * lp)
            return 0
        lax.fori_loop(0, C // L, _group, 0)

        def _pair(i, _):
            lane_i = jnp.full((L,), i, jnp.int32)
            c1 = plsc.load_gather(c1_v, [lane_i])
            c2 = plsc.load_gather(c2_v, [lane_i])
            for q in range(D // L):
                sl = pl.ds(q * L, L)
                CU_v[i, sl] = c1 * B_v[i, sl] + c2 * PB_v[i, sl]
                CI_v[i, sl] = c1 * A_v[i, sl] + c2 * PA_v[i, sl]
            return 0
        lax.fori_loop(0, C, _pair, 0)

        pltpu.sync_copy(CU_v, accu_sh.at[saidx_v], add=True)
        pltpu.sync_copy(CI_v, acci_sh.at[sbidx_v], add=True)
        return 0

    lax.fori_loop(0, CHUNKS, _chunk, 0)
    plsc.subcore_barrier()

    pltpu.sync_copy(accu_sh.at[pl.ds(sid * ROWS_PER_TILE, ROWS_PER_TILE)],
                    acc_out.at[cid, 0, pl.ds(sid * ROWS_PER_TILE, ROWS_PER_TILE)])
    pltpu.sync_copy(acci_sh.at[pl.ds(sid * ROWS_PER_TILE, ROWS_PER_TILE)],
                    acc_out.at[cid, 1, pl.ds(sid * ROWS_PER_TILE, ROWS_PER_TILE)])


_sc_update = functools.partial(
    pl.kernel,
    out_type=jax.ShapeDtypeStruct((NC, 2, NU, D), jnp.float32),
    mesh=plsc.VectorSubcoreMesh(core_axis_name="c", subcore_axis_name="s"),
    compiler_params=pltpu.CompilerParams(needs_layout_passes=False,
                                         use_tc_tiling_on_sc=False),
    scratch_types=[
        pltpu.VMEM((C,), jnp.int32),
        pltpu.VMEM((C,), jnp.int32),
        pltpu.VMEM((C,), jnp.int32),
        pltpu.VMEM((C,), jnp.int32),
        pltpu.VMEM((C,), jnp.int32),
        pltpu.VMEM((C,), jnp.int32),
        pltpu.VMEM((C,), jnp.float32),
        pltpu.VMEM((C,), jnp.float32),
        pltpu.VMEM((C,), jnp.float32),
        pltpu.VMEM((C,), jnp.float32),
        pltpu.VMEM((C,), jnp.float32),
        pltpu.VMEM((C,), jnp.float32),
        pltpu.VMEM((C, D), jnp.float32),
        pltpu.VMEM((C, D), jnp.float32),
        pltpu.VMEM((C, D), jnp.float32),
        pltpu.VMEM((C, D), jnp.float32),
        pltpu.VMEM((C, D), jnp.float32),
        pltpu.VMEM((C, D), jnp.float32),
        pltpu.VMEM_SHARED((NU + 1, D), jnp.float32),
        pltpu.VMEM_SHARED((NI + 1, D), jnp.float32),
        pltpu.SemaphoreType.DMA,
    ],
)(_sc_body)


R_BLK = 5000
N_BLK = 100000 // R_BLK


def _tc_body(u_ref, i_ref, acc_ref, p_ref, o_ref):
    o_ref[0] = u_ref[...]
    o_ref[1] = i_ref[...]

    @pl.when(pl.program_id(0) == 0)
    def _():
        o_ref[0, 0:NU, :] = (o_ref[0, 0:NU, :] + acc_ref[0, 0] + acc_ref[1, 0]
                             + (1.0 / T) * p_ref[0:NU, :])
        o_ref[1, 0:NI, :] = (o_ref[1, 0:NI, :] + acc_ref[0, 1] + acc_ref[1, 1]
                             + (1.0 / T) * p_ref[NU:NU + NI, :])


def kernel(user_emb, item_emb, p, train_labels, unlearn_labels,
           nei_users, nei_items, train_users, train_items,
           unlearn_users, unlearn_items):
    n_rows = user_emb.shape[0]
    pad = P_TOT - T - U
    p_mat = p.reshape(NU + NI, D)
    p_pad = jnp.concatenate([p_mat, jnp.zeros((1, D), jnp.float32)], axis=0)
    idx_a = jnp.concatenate([train_users, unlearn_users,
                             jnp.full((pad,), NU, jnp.int32)])
    idx_b = jnp.concatenate([train_items, unlearn_items,
                             jnp.full((pad,), NI, jnp.int32)])
    y = jnp.concatenate([train_labels, unlearn_labels,
                         jnp.zeros((pad,), jnp.float32)])
    kind = jnp.concatenate([jnp.zeros((T,), jnp.float32),
                            jnp.ones((U + pad,), jnp.float32)])

    acc = _sc_update(user_emb, item_emb, p_pad, idx_a, idx_b, y, kind)

    out = pl.pallas_call(
        _tc_body,
        grid=(N_BLK,),
        in_specs=[
            pl.BlockSpec((R_BLK, D), lambda j: (j, 0)),
            pl.BlockSpec((R_BLK, D), lambda j: (j, 0)),
            pl.BlockSpec((NC, 2, NU, D), lambda j: (0, 0, 0, 0)),
            pl.BlockSpec((NU + NI, D), lambda j: (0, 0)),
        ],
        out_specs=pl.BlockSpec((2, R_BLK, D), lambda j: (0, j, 0)),
        out_shape=jax.ShapeDtypeStruct((2, n_rows, D), jnp.float32),
    )(user_emb, item_emb, acc, p_mat)
    return out
